# Initial kernel scaffold; baseline (speedup 1.0000x reference)
import jax
import jax.numpy as jnp
from jax.experimental import pallas as pl


def kernel(xyz, W1, b1, g1, be1, W2, b2, g2, be2, W3, b3, g3, be3, Wfc, bfc):
    raise NotImplementedError("write your pallas kernel here")


# speedup vs baseline: 8.6536x; 8.6536x over previous
import functools

import jax
import jax.numpy as jnp
from jax.experimental import pallas as pl
from jax.experimental.pallas import tpu as pltpu

B = 16
N1, S1, K1, R1 = 4096, 512, 32, 0.2
S2, K2, R2 = 128, 64, 0.4
C1T = 16
C2T = 144
C3T = 272
CNT1 = B * S1 * K1
CNT2 = B * S2 * K2
CNT3 = B * S2


def _fps_body(npoint, x_ref, y_ref, z_ref, cx_ref, cy_ref, cz_ref):
    b, n = x_ref.shape
    xs = x_ref[...]
    ys = y_ref[...]
    zs = z_ref[...]
    iota_n = jax.lax.broadcasted_iota(jnp.int32, (b, n), 1)
    iota_s = jax.lax.broadcasted_iota(jnp.int32, (b, npoint), 1)

    def body(i, carry):
        dist, far, cxh, cyh, czh = carry
        mask = iota_n == far
        cx = jnp.sum(jnp.where(mask, xs, 0.0), axis=1, keepdims=True)
        cy = jnp.sum(jnp.where(mask, ys, 0.0), axis=1, keepdims=True)
        cz = jnp.sum(jnp.where(mask, zs, 0.0), axis=1, keepdims=True)
        hit = iota_s == i
        cxh = jnp.where(hit, cx, cxh)
        cyh = jnp.where(hit, cy, cyh)
        czh = jnp.where(hit, cz, czh)
        dx = xs - cx
        dy = ys - cy
        dz = zs - cz
        d = (dx * dx + dy * dy) + dz * dz
        dist = jnp.minimum(dist, d)
        m = jnp.max(dist, axis=1, keepdims=True)
        far = jnp.min(jnp.where(dist == m, iota_n, n), axis=1, keepdims=True)
        return dist, far, cxh, cyh, czh

    init = (
        jnp.full((b, n), 1e10, jnp.float32),
        jnp.zeros((b, 1), jnp.int32),
        jnp.zeros((b, npoint), jnp.float32),
        jnp.zeros((b, npoint), jnp.float32),
        jnp.zeros((b, npoint), jnp.float32),
    )
    _, _, cxh, cyh, czh = jax.lax.fori_loop(0, npoint, body, init)
    cx_ref[...] = cxh
    cy_ref[...] = cyh
    cz_ref[...] = czh


def _fps(xs, ys, zs, npoint):
    b, n = xs.shape
    out = jax.ShapeDtypeStruct((b, npoint), jnp.float32)
    return pl.pallas_call(
        functools.partial(_fps_body, npoint),
        out_shape=[out, out, out],
    )(xs, ys, zs)


def _bq_body(K, r2, npts, cx_ref, cy_ref, cz_ref, x_ref, y_ref, z_ref,
             out_ref):
    s = cx_ref.shape[1]
    n = x_ref.shape[2]
    cxs = cx_ref[0]
    cys = cy_ref[0]
    czs = cz_ref[0]
    px = x_ref[0]
    py = y_ref[0]
    pz = z_ref[0]
    bf = jnp.bfloat16
    bcx = cxs.astype(bf).astype(jnp.float32)
    bcy = cys.astype(bf).astype(jnp.float32)
    bcz = czs.astype(bf).astype(jnp.float32)
    bpx = px.astype(bf).astype(jnp.float32)
    bpy = py.astype(bf).astype(jnp.float32)
    bpz = pz.astype(bf).astype(jnp.float32)
    cross = (bcx * bpx + bcz * bpz) + bcy * bpy
    s2 = (cxs * cxs + czs * czs) + cys * cys
    p2 = (px * px + pz * pz) + py * py
    d = (-2.0 * cross + s2) + p2
    iota = jax.lax.broadcasted_iota(jnp.int32, (s, n), 1)
    cand = jnp.where(jnp.logical_not(d > r2), iota, n)
    prev = jnp.full((s, 1), -1, jnp.int32)
    first = None
    cols = []
    for k in range(K):
        val = jnp.where(cand > prev, cand, n)
        cur = jnp.min(val, axis=1, keepdims=True)
        if k == 0:
            first = cur
        cols.append(jnp.where(cur == n, first, cur))
        prev = cur
    out = jnp.concatenate(cols, axis=1)
    out_ref[0] = out + pl.program_id(0) * npts


def _ball_query(cxT, cyT, czT, xs, ys, zs, K, radius):
    s = cxT.shape[1]
    b, n = xs.shape
    cspec = pl.BlockSpec((1, s, 1), lambda i: (i, 0, 0))
    pspec = pl.BlockSpec((1, 1, n), lambda i: (i, 0, 0))
    return pl.pallas_call(
        functools.partial(_bq_body, K, radius * radius, n),
        grid=(b,),
        in_specs=[cspec, cspec, cspec, pspec, pspec, pspec],
        out_specs=pl.BlockSpec((1, s, K), lambda i: (i, 0, 0)),
        out_shape=jax.ShapeDtypeStruct((b, s, K), jnp.int32),
    )(cxT, cyT, czT, xs[:, None, :], ys[:, None, :], zs[:, None, :])


def _mlp_body(S, K, centered, g_ref, wt_ref, bias_ref,
              cx_ref, cy_ref, cz_ref,
              pmax_ref, pmin_ref, sh_ref, sh2_ref):
    c = g_ref.shape[2]
    co = wt_ref.shape[1]
    g3 = g_ref[0].reshape(S, K, c)
    if centered:
        ctr = jnp.concatenate(
            [cx_ref[0], cy_ref[0], cz_ref[0],
             jnp.zeros((S, c - 3), jnp.float32)], axis=1)
        g3 = g3 - ctr[:, None, :]
    gb = g3.astype(jnp.bfloat16).reshape(S * K, c)
    wb = wt_ref[...].astype(jnp.bfloat16)
    h = jnp.dot(gb, wb, preferred_element_type=jnp.float32)
    h3 = h.reshape(S, K, co) + bias_ref[...][:, None, :]
    pmax_ref[0] = jnp.max(h3, axis=1)
    pmin_ref[0] = jnp.min(h3, axis=1)
    sh = jnp.sum(jnp.sum(h3, axis=1), axis=0, keepdims=True)
    sq = h3 * h3
    sh2 = jnp.sum(jnp.sum(sq, axis=1), axis=0, keepdims=True)

    @pl.when(pl.program_id(0) == 0)
    def _():
        sh_ref[...] = jnp.zeros_like(sh_ref)
        sh2_ref[...] = jnp.zeros_like(sh2_ref)

    sh_ref[...] += sh
    sh2_ref[...] += sh2


def _mlp_pool(g, wt, bias, cxT, cyT, czT, S, K, centered):
    b = g.shape[0]
    c = g.shape[2]
    co = wt.shape[1]
    cspec = pl.BlockSpec((1, S, 1), lambda i: (i, 0, 0))
    acc = pl.BlockSpec((1, co), lambda i: (0, 0))
    return pl.pallas_call(
        functools.partial(_mlp_body, S, K, centered),
        grid=(b,),
        in_specs=[
            pl.BlockSpec((1, S * K, c), lambda i: (i, 0, 0)),
            pl.BlockSpec((c, co), lambda i: (0, 0)),
            pl.BlockSpec((1, co), lambda i: (0, 0)),
            cspec, cspec, cspec,
        ],
        out_specs=[
            pl.BlockSpec((1, S, co), lambda i: (i, 0, 0)),
            pl.BlockSpec((1, S, co), lambda i: (i, 0, 0)),
            acc, acc,
        ],
        out_shape=[
            jax.ShapeDtypeStruct((b, S, co), jnp.float32),
            jax.ShapeDtypeStruct((b, S, co), jnp.float32),
            jax.ShapeDtypeStruct((1, co), jnp.float32),
            jax.ShapeDtypeStruct((1, co), jnp.float32),
        ],
    )(g, wt, bias, cxT, cyT, czT)


def _bn_body(cnt, ct, pmax_ref, pmin_ref, sh_ref, sh2_ref, gam_ref,
             bet_ref, cx_ref, cy_ref, cz_ref, tbl_ref):
    s, co = pmax_ref.shape[1], pmax_ref.shape[2]
    mu = sh_ref[...] * (1.0 / cnt)
    var = jnp.maximum(sh2_ref[...] * (1.0 / cnt) - mu * mu, 0.0)
    inv = 1.0 / jnp.sqrt(var + 1e-5)
    a = inv * gam_ref[...]
    cc = bet_ref[...] - mu * a
    sel = jnp.where(a >= 0.0, pmax_ref[0], pmin_ref[0])
    y = jnp.maximum(sel * a + cc, 0.0)
    tbl = jnp.concatenate(
        [cx_ref[0], cy_ref[0], cz_ref[0], y,
         jnp.zeros((s, ct - 3 - co), jnp.float32)], axis=1)
    tbl_ref[0] = tbl


def _bn_table(pmax, pmin, sh, sh2, gam, bet, cxT, cyT, czT, cnt, ct):
    b, s, co = pmax.shape
    cspec = pl.BlockSpec((1, s, 1), lambda i: (i, 0, 0))
    stat = pl.BlockSpec((1, co), lambda i: (0, 0))
    return pl.pallas_call(
        functools.partial(_bn_body, cnt, ct),
        grid=(b,),
        in_specs=[
            pl.BlockSpec((1, s, co), lambda i: (i, 0, 0)),
            pl.BlockSpec((1, s, co), lambda i: (i, 0, 0)),
            stat, stat, stat, stat,
            cspec, cspec, cspec,
        ],
        out_specs=pl.BlockSpec((1, s, ct), lambda i: (i, 0, 0)),
        out_shape=jax.ShapeDtypeStruct((b, s, ct), jnp.float32),
    )(pmax, pmin, sh, sh2, gam, bet, cxT, cyT, czT)


def _final_body(cnt, pmax_ref, pmin_ref, sh_ref, sh2_ref, gam_ref, bet_ref,
                wfct_ref, bfc_ref, out_ref):
    mu = sh_ref[...] * (1.0 / cnt)
    var = jnp.maximum(sh2_ref[...] * (1.0 / cnt) - mu * mu, 0.0)
    inv = 1.0 / jnp.sqrt(var + 1e-5)
    a = inv * gam_ref[...]
    cc = bet_ref[...] - mu * a
    sel = jnp.where(a >= 0.0, pmax_ref[:, 0, :], pmin_ref[:, 0, :])
    y = jnp.maximum(sel * a + cc, 0.0)
    out_ref[...] = (jnp.dot(y.astype(jnp.bfloat16),
                            wfct_ref[...].astype(jnp.bfloat16),
                            preferred_element_type=jnp.float32)
                    + bfc_ref[...])


def _final(pmax, pmin, sh, sh2, gam, bet, wfct, bfc):
    b = pmax.shape[0]
    cf = wfct.shape[1]
    return pl.pallas_call(
        functools.partial(_final_body, CNT3),
        out_shape=jax.ShapeDtypeStruct((b, cf), jnp.float32),
    )(pmax, pmin, sh, sh2, gam, bet, wfct, bfc)


def _gather_rows(tbl2d, idx):
    b, s, k = idx.shape
    g = jnp.take(tbl2d, idx.reshape(b, s * k), axis=0)
    return g


def kernel(xyz, W1, b1, g1, be1, W2, b2, g2, be2, W3, b3, g3, be3, Wfc, bfc):
    f32 = jnp.float32
    xs = xyz[:, :, 0]
    ys = xyz[:, :, 1]
    zs = xyz[:, :, 2]

    c1x, c1y, c1z = _fps(xs, ys, zs, S1)
    c1xT, c1yT, c1zT = c1x[:, :, None], c1y[:, :, None], c1z[:, :, None]
    idx1 = _ball_query(c1xT, c1yT, c1zT, xs, ys, zs, K1, R1)

    tbl0 = jnp.zeros((B, N1, C1T), f32).at[:, :, 0].set(xs)
    tbl0 = tbl0.at[:, :, 1].set(ys).at[:, :, 2].set(zs)
    g1rows = _gather_rows(tbl0.reshape(B * N1, C1T), idx1)

    wt1 = jnp.zeros((C1T, 128), f32).at[0:3, :].set(W1.T)
    p1max, p1min, sh1, sh1sq = _mlp_pool(
        g1rows, wt1, b1[None, :], c1xT, c1yT, c1zT, S1, K1, True)
    tbl1 = _bn_table(p1max, p1min, sh1, sh1sq, g1[None, :], be1[None, :],
                     c1xT, c1yT, c1zT, CNT1, C2T)

    c2x, c2y, c2z = _fps(c1x, c1y, c1z, S2)
    c2xT, c2yT, c2zT = c2x[:, :, None], c2y[:, :, None], c2z[:, :, None]
    idx2 = _ball_query(c2xT, c2yT, c2zT, c1x, c1y, c1z, K2, R2)
    g2rows = _gather_rows(tbl1.reshape(B * S1, C2T), idx2)

    wt2 = jnp.zeros((C2T, 256), f32).at[0:131, :].set(W2.T)
    p2max, p2min, sh2, sh2sq = _mlp_pool(
        g2rows, wt2, b2[None, :], c2xT, c2yT, c2zT, S2, K2, True)
    tbl2 = _bn_table(p2max, p2min, sh2, sh2sq, g2[None, :], be2[None, :],
                     c2xT, c2yT, c2zT, CNT2, C3T)

    wt3 = jnp.zeros((C3T, 512), f32).at[0:259, :].set(W3.T)
    zcol = jnp.zeros((B, 1, 1), f32)
    p3max, p3min, sh3, sh3sq = _mlp_pool(
        tbl2, wt3, b3[None, :], zcol, zcol, zcol, 1, S2, False)
    out = _final(p3max, p3min, sh3, sh3sq, g3[None, :], be3[None, :],
                 Wfc.T, bfc[None, :])
    return out


# speedup vs baseline: 12.8631x; 1.4865x over previous
import functools

import jax
import jax.numpy as jnp
from jax import lax
from jax.experimental import pallas as pl
from jax.experimental.pallas import tpu as pltpu
from jax.experimental.pallas import tpu_sc as plsc

B = 16
N1, S1, K1, R1 = 4096, 512, 32, 0.2
S2, K2, R2 = 128, 64, 0.4
C1T = 16
C2T = 144
C3T = 272
CNT1 = B * S1 * K1
CNT2 = B * S2 * K2
CNT3 = B * S2


def _fps_body(npoint, x_ref, y_ref, z_ref, cx_ref, cy_ref, cz_ref):
    b, n = x_ref.shape
    xs = x_ref[...]
    ys = y_ref[...]
    zs = z_ref[...]
    iota_n = jax.lax.broadcasted_iota(jnp.int32, (b, n), 1)
    iota_s = jax.lax.broadcasted_iota(jnp.int32, (b, npoint), 1)

    def body(i, carry):
        dist, far, cxh, cyh, czh = carry
        mask = iota_n == far
        cx = jnp.sum(jnp.where(mask, xs, 0.0), axis=1, keepdims=True)
        cy = jnp.sum(jnp.where(mask, ys, 0.0), axis=1, keepdims=True)
        cz = jnp.sum(jnp.where(mask, zs, 0.0), axis=1, keepdims=True)
        hit = iota_s == i
        cxh = jnp.where(hit, cx, cxh)
        cyh = jnp.where(hit, cy, cyh)
        czh = jnp.where(hit, cz, czh)
        dx = xs - cx
        dy = ys - cy
        dz = zs - cz
        d = (dx * dx + dy * dy) + dz * dz
        dist = jnp.minimum(dist, d)
        m = jnp.max(dist, axis=1, keepdims=True)
        far = jnp.min(jnp.where(dist == m, iota_n, n), axis=1, keepdims=True)
        return dist, far, cxh, cyh, czh

    init = (
        jnp.full((b, n), 1e10, jnp.float32),
        jnp.zeros((b, 1), jnp.int32),
        jnp.zeros((b, npoint), jnp.float32),
        jnp.zeros((b, npoint), jnp.float32),
        jnp.zeros((b, npoint), jnp.float32),
    )
    _, _, cxh, cyh, czh = jax.lax.fori_loop(0, npoint, body, init)
    cx_ref[...] = cxh
    cy_ref[...] = cyh
    cz_ref[...] = czh


def _fps(xs, ys, zs, npoint):
    b, n = xs.shape
    out = jax.ShapeDtypeStruct((b, npoint), jnp.float32)
    return pl.pallas_call(
        functools.partial(_fps_body, npoint),
        out_shape=[out, out, out],
    )(xs, ys, zs)


def _bq_body(K, r2, npts, cx_ref, cy_ref, cz_ref, x_ref, y_ref, z_ref,
             out_ref):
    s = cx_ref.shape[1]
    n = x_ref.shape[2]
    cxs = cx_ref[0]
    cys = cy_ref[0]
    czs = cz_ref[0]
    px = x_ref[0]
    py = y_ref[0]
    pz = z_ref[0]
    bf = jnp.bfloat16
    bcx = cxs.astype(bf).astype(jnp.float32)
    bcy = cys.astype(bf).astype(jnp.float32)
    bcz = czs.astype(bf).astype(jnp.float32)
    bpx = px.astype(bf).astype(jnp.float32)
    bpy = py.astype(bf).astype(jnp.float32)
    bpz = pz.astype(bf).astype(jnp.float32)
    cross = (bcx * bpx + bcz * bpz) + bcy * bpy
    s2 = (cxs * cxs + czs * czs) + cys * cys
    p2 = (px * px + pz * pz) + py * py
    d = (-2.0 * cross + s2) + p2
    iota = jax.lax.broadcasted_iota(jnp.int32, (s, n), 1)
    cand = jnp.where(jnp.logical_not(d > r2), iota, n)
    prev = jnp.full((s, 1), -1, jnp.int32)
    first = None
    cols = []
    for k in range(K):
        val = jnp.where(cand > prev, cand, n)
        cur = jnp.min(val, axis=1, keepdims=True)
        if k == 0:
            first = cur
        cols.append(jnp.where(cur == n, first, cur))
        prev = cur
    out = jnp.concatenate(cols, axis=1)
    out = jnp.minimum(out, n - 1)
    out_ref[0] = out + pl.program_id(0) * npts


def _ball_query(cxT, cyT, czT, xs, ys, zs, K, radius):
    s = cxT.shape[1]
    b, n = xs.shape
    cspec = pl.BlockSpec((1, s, 1), lambda i: (i, 0, 0))
    pspec = pl.BlockSpec((1, 1, n), lambda i: (i, 0, 0))
    return pl.pallas_call(
        functools.partial(_bq_body, K, radius * radius, n),
        grid=(b,),
        in_specs=[cspec, cspec, cspec, pspec, pspec, pspec],
        out_specs=pl.BlockSpec((1, s, K), lambda i: (i, 0, 0)),
        out_shape=jax.ShapeDtypeStruct((b, s, K), jnp.int32),
    )(cxT, cyT, czT, xs[:, None, :], ys[:, None, :], zs[:, None, :])


def _mlp_body(S, K, centered, g_ref, wt_ref, bias_ref,
              cx_ref, cy_ref, cz_ref,
              pmax_ref, pmin_ref, sh_ref, sh2_ref):
    c = g_ref.shape[2]
    co = wt_ref.shape[1]
    g3 = g_ref[0].reshape(S, K, c)
    if centered:
        ctr = jnp.concatenate(
            [cx_ref[0], cy_ref[0], cz_ref[0],
             jnp.zeros((S, c - 3), jnp.float32)], axis=1)
        g3 = g3 - ctr[:, None, :]
    gb = g3.astype(jnp.bfloat16).reshape(S * K, c)
    wb = wt_ref[...].astype(jnp.bfloat16)
    h = jnp.dot(gb, wb, preferred_element_type=jnp.float32)
    h3 = h.reshape(S, K, co) + bias_ref[...][:, None, :]
    pmax_ref[0] = jnp.max(h3, axis=1)
    pmin_ref[0] = jnp.min(h3, axis=1)
    sh = jnp.sum(jnp.sum(h3, axis=1), axis=0, keepdims=True)
    sq = h3 * h3
    sh2 = jnp.sum(jnp.sum(sq, axis=1), axis=0, keepdims=True)

    @pl.when(pl.program_id(0) == 0)
    def _():
        sh_ref[...] = jnp.zeros_like(sh_ref)
        sh2_ref[...] = jnp.zeros_like(sh2_ref)

    sh_ref[...] += sh
    sh2_ref[...] += sh2


def _mlp_pool(g, wt, bias, cxT, cyT, czT, S, K, centered):
    b = g.shape[0]
    c = g.shape[2]
    co = wt.shape[1]
    cspec = pl.BlockSpec((1, S, 1), lambda i: (i, 0, 0))
    acc = pl.BlockSpec((1, co), lambda i: (0, 0))
    return pl.pallas_call(
        functools.partial(_mlp_body, S, K, centered),
        grid=(b,),
        in_specs=[
            pl.BlockSpec((1, S * K, c), lambda i: (i, 0, 0)),
            pl.BlockSpec((c, co), lambda i: (0, 0)),
            pl.BlockSpec((1, co), lambda i: (0, 0)),
            cspec, cspec, cspec,
        ],
        out_specs=[
            pl.BlockSpec((1, S, co), lambda i: (i, 0, 0)),
            pl.BlockSpec((1, S, co), lambda i: (i, 0, 0)),
            acc, acc,
        ],
        out_shape=[
            jax.ShapeDtypeStruct((b, S, co), jnp.float32),
            jax.ShapeDtypeStruct((b, S, co), jnp.float32),
            jax.ShapeDtypeStruct((1, co), jnp.float32),
            jax.ShapeDtypeStruct((1, co), jnp.float32),
        ],
    )(g, wt, bias, cxT, cyT, czT)


def _bn_body(cnt, ct, pmax_ref, pmin_ref, sh_ref, sh2_ref, gam_ref,
             bet_ref, cx_ref, cy_ref, cz_ref, tbl_ref):
    s, co = pmax_ref.shape[1], pmax_ref.shape[2]
    mu = sh_ref[...] * (1.0 / cnt)
    var = jnp.maximum(sh2_ref[...] * (1.0 / cnt) - mu * mu, 0.0)
    inv = 1.0 / jnp.sqrt(var + 1e-5)
    a = inv * gam_ref[...]
    cc = bet_ref[...] - mu * a
    sel = jnp.where(a >= 0.0, pmax_ref[0], pmin_ref[0])
    y = jnp.maximum(sel * a + cc, 0.0)
    tbl = jnp.concatenate(
        [cx_ref[0], cy_ref[0], cz_ref[0], y,
         jnp.zeros((s, ct - 3 - co), jnp.float32)], axis=1)
    tbl_ref[0] = tbl


def _bn_table(pmax, pmin, sh, sh2, gam, bet, cxT, cyT, czT, cnt, ct):
    b, s, co = pmax.shape
    cspec = pl.BlockSpec((1, s, 1), lambda i: (i, 0, 0))
    stat = pl.BlockSpec((1, co), lambda i: (0, 0))
    return pl.pallas_call(
        functools.partial(_bn_body, cnt, ct),
        grid=(b,),
        in_specs=[
            pl.BlockSpec((1, s, co), lambda i: (i, 0, 0)),
            pl.BlockSpec((1, s, co), lambda i: (i, 0, 0)),
            stat, stat, stat, stat,
            cspec, cspec, cspec,
        ],
        out_specs=pl.BlockSpec((1, s, ct), lambda i: (i, 0, 0)),
        out_shape=jax.ShapeDtypeStruct((b, s, ct), jnp.float32),
    )(pmax, pmin, sh, sh2, gam, bet, cxT, cyT, czT)


def _final_body(cnt, pmax_ref, pmin_ref, sh_ref, sh2_ref, gam_ref, bet_ref,
                wfct_ref, bfc_ref, out_ref):
    mu = sh_ref[...] * (1.0 / cnt)
    var = jnp.maximum(sh2_ref[...] * (1.0 / cnt) - mu * mu, 0.0)
    inv = 1.0 / jnp.sqrt(var + 1e-5)
    a = inv * gam_ref[...]
    cc = bet_ref[...] - mu * a
    sel = jnp.where(a >= 0.0, pmax_ref[:, 0, :], pmin_ref[:, 0, :])
    y = jnp.maximum(sel * a + cc, 0.0)
    out_ref[...] = (jnp.dot(y.astype(jnp.bfloat16),
                            wfct_ref[...].astype(jnp.bfloat16),
                            preferred_element_type=jnp.float32)
                    + bfc_ref[...])


def _final(pmax, pmin, sh, sh2, gam, bet, wfct, bfc):
    b = pmax.shape[0]
    cf = wfct.shape[1]
    return pl.pallas_call(
        functools.partial(_final_body, CNT3),
        out_shape=jax.ShapeDtypeStruct((b, cf), jnp.float32),
    )(pmax, pmin, sh, sh2, gam, bet, wfct, bfc)


def _gather_rows(tbl2d, idx, chunk=128):
    b, s, k = idx.shape
    m = b * s * k
    c = tbl2d.shape[1]
    info = plsc.get_sparse_core_info()
    nw = info.num_cores * info.num_subcores
    per_w = m // nw
    nch = per_w // chunk
    assert per_w * nw == m and nch * chunk == per_w
    mesh = plsc.VectorSubcoreMesh(core_axis_name="c", subcore_axis_name="s")

    @functools.partial(
        pl.kernel, mesh=mesh,
        out_type=jax.ShapeDtypeStruct((m, c), jnp.float32),
        compiler_params=pltpu.CompilerParams(use_tc_tiling_on_sc=False),
        scratch_types=[
            pltpu.VMEM((chunk,), jnp.int32),
            pltpu.VMEM((chunk, c), jnp.float32),
            pltpu.SemaphoreType.DMA,
        ],
    )
    def kern(tbl_hbm, idx_hbm, out_hbm, idx_v, rows_v, sem):
        wid = lax.axis_index("s") * info.num_cores + lax.axis_index("c")
        base = wid * per_w

        def body(j, carry):
            off = base + j * chunk
            pltpu.sync_copy(idx_hbm.at[pl.ds(off, chunk)], idx_v)
            pltpu.async_copy(tbl_hbm.at[idx_v], rows_v, sem).wait()
            pltpu.sync_copy(rows_v, out_hbm.at[pl.ds(off, chunk)])
            return carry

        lax.fori_loop(0, nch, body, 0)

    out = kern(tbl2d, idx.reshape(m))
    return out.reshape(b, s * k, c)


def kernel(xyz, W1, b1, g1, be1, W2, b2, g2, be2, W3, b3, g3, be3, Wfc, bfc):
    f32 = jnp.float32
    xs = xyz[:, :, 0]
    ys = xyz[:, :, 1]
    zs = xyz[:, :, 2]

    c1x, c1y, c1z = _fps(xs, ys, zs, S1)
    c1xT, c1yT, c1zT = c1x[:, :, None], c1y[:, :, None], c1z[:, :, None]
    idx1 = _ball_query(c1xT, c1yT, c1zT, xs, ys, zs, K1, R1)

    tbl0 = jnp.zeros((B, N1, C1T), f32).at[:, :, 0].set(xs)
    tbl0 = tbl0.at[:, :, 1].set(ys).at[:, :, 2].set(zs)
    g1rows = _gather_rows(tbl0.reshape(B * N1, C1T), idx1)

    wt1 = jnp.zeros((C1T, 128), f32).at[0:3, :].set(W1.T)
    p1max, p1min, sh1, sh1sq = _mlp_pool(
        g1rows, wt1, b1[None, :], c1xT, c1yT, c1zT, S1, K1, True)
    tbl1 = _bn_table(p1max, p1min, sh1, sh1sq, g1[None, :], be1[None, :],
                     c1xT, c1yT, c1zT, CNT1, C2T)

    c2x, c2y, c2z = _fps(c1x, c1y, c1z, S2)
    c2xT, c2yT, c2zT = c2x[:, :, None], c2y[:, :, None], c2z[:, :, None]
    idx2 = _ball_query(c2xT, c2yT, c2zT, c1x, c1y, c1z, K2, R2)
    g2rows = _gather_rows(tbl1.reshape(B * S1, C2T), idx2)

    wt2 = jnp.zeros((C2T, 256), f32).at[0:131, :].set(W2.T)
    p2max, p2min, sh2, sh2sq = _mlp_pool(
        g2rows, wt2, b2[None, :], c2xT, c2yT, c2zT, S2, K2, True)
    tbl2 = _bn_table(p2max, p2min, sh2, sh2sq, g2[None, :], be2[None, :],
                     c2xT, c2yT, c2zT, CNT2, C3T)

    wt3 = jnp.zeros((C3T, 512), f32).at[0:259, :].set(W3.T)
    zcol = jnp.zeros((B, 1, 1), f32)
    p3max, p3min, sh3, sh3sq = _mlp_pool(
        tbl2, wt3, b3[None, :], zcol, zcol, zcol, 1, S2, False)
    out = _final(p3max, p3min, sh3, sh3sq, g3[None, :], be3[None, :],
                 Wfc.T, bfc[None, :])
    return out


# speedup vs baseline: 13.4052x; 1.0421x over previous
import functools

import jax
import jax.numpy as jnp
from jax import lax
from jax.experimental import pallas as pl
from jax.experimental.pallas import tpu as pltpu
from jax.experimental.pallas import tpu_sc as plsc

B = 16
N1, S1, K1, R1 = 4096, 512, 32, 0.2
S2, K2, R2 = 128, 64, 0.4
C1T = 16
C2T = 144
C3T = 272
CNT1 = B * S1 * K1
CNT2 = B * S2 * K2
CNT3 = B * S2


def _fps_body(npoint, x_ref, y_ref, z_ref, cx_ref, cy_ref, cz_ref):
    b, n = x_ref.shape
    xs = x_ref[...]
    ys = y_ref[...]
    zs = z_ref[...]
    iota_n = jax.lax.broadcasted_iota(jnp.int32, (b, n), 1)
    iota_s = jax.lax.broadcasted_iota(jnp.int32, (b, npoint), 1)

    def body(i, carry):
        dist, far, cxh, cyh, czh = carry
        mask = iota_n == far
        cx = jnp.sum(jnp.where(mask, xs, 0.0), axis=1, keepdims=True)
        cy = jnp.sum(jnp.where(mask, ys, 0.0), axis=1, keepdims=True)
        cz = jnp.sum(jnp.where(mask, zs, 0.0), axis=1, keepdims=True)
        hit = iota_s == i
        cxh = jnp.where(hit, cx, cxh)
        cyh = jnp.where(hit, cy, cyh)
        czh = jnp.where(hit, cz, czh)
        dx = xs - cx
        dy = ys - cy
        dz = zs - cz
        d = (dx * dx + dy * dy) + dz * dz
        dist = jnp.minimum(dist, d)
        m = jnp.max(dist, axis=1, keepdims=True)
        far = jnp.min(jnp.where(dist == m, iota_n, n), axis=1, keepdims=True)
        return dist, far, cxh, cyh, czh

    init = (
        jnp.full((b, n), 1e10, jnp.float32),
        jnp.zeros((b, 1), jnp.int32),
        jnp.zeros((b, npoint), jnp.float32),
        jnp.zeros((b, npoint), jnp.float32),
        jnp.zeros((b, npoint), jnp.float32),
    )
    _, _, cxh, cyh, czh = jax.lax.fori_loop(0, npoint, body, init)
    cx_ref[...] = cxh
    cy_ref[...] = cyh
    cz_ref[...] = czh


def _fps(xs, ys, zs, npoint):
    b, n = xs.shape
    out = jax.ShapeDtypeStruct((b, npoint), jnp.float32)
    return pl.pallas_call(
        functools.partial(_fps_body, npoint),
        out_shape=[out, out, out],
    )(xs, ys, zs)


def _bq_body(K, r2, npts, cx_ref, cy_ref, cz_ref, x_ref, y_ref, z_ref,
             out_ref):
    s = cx_ref.shape[1]
    n = x_ref.shape[2]
    cxs = cx_ref[0]
    cys = cy_ref[0]
    czs = cz_ref[0]
    px = x_ref[0]
    py = y_ref[0]
    pz = z_ref[0]
    bf = jnp.bfloat16
    bcx = cxs.astype(bf).astype(jnp.float32)
    bcy = cys.astype(bf).astype(jnp.float32)
    bcz = czs.astype(bf).astype(jnp.float32)
    bpx = px.astype(bf).astype(jnp.float32)
    bpy = py.astype(bf).astype(jnp.float32)
    bpz = pz.astype(bf).astype(jnp.float32)
    cross = (bcx * bpx + bcz * bpz) + bcy * bpy
    s2 = (cxs * cxs + czs * czs) + cys * cys
    p2 = (px * px + pz * pz) + py * py
    d = (-2.0 * cross + s2) + p2
    iota = jax.lax.broadcasted_iota(jnp.int32, (s, n), 1)
    cand = jnp.where(jnp.logical_not(d > r2), iota, n)
    prev = jnp.full((s, 1), -1, jnp.int32)
    first = None
    cols = []
    for k in range(K):
        val = jnp.where(cand > prev, cand, n)
        cur = jnp.min(val, axis=1, keepdims=True)
        if k == 0:
            first = cur
        cols.append(jnp.where(cur == n, first, cur))
        prev = cur
    out = jnp.concatenate(cols, axis=1)
    out = jnp.minimum(out, n - 1)
    out_ref[0] = out + pl.program_id(0) * npts


def _ball_query(cxT, cyT, czT, xs, ys, zs, K, radius):
    s = cxT.shape[1]
    b, n = xs.shape
    cspec = pl.BlockSpec((1, s, 1), lambda i: (i, 0, 0))
    pspec = pl.BlockSpec((1, 1, n), lambda i: (i, 0, 0))
    return pl.pallas_call(
        functools.partial(_bq_body, K, radius * radius, n),
        grid=(b,),
        in_specs=[cspec, cspec, cspec, pspec, pspec, pspec],
        out_specs=pl.BlockSpec((1, s, K), lambda i: (i, 0, 0)),
        out_shape=jax.ShapeDtypeStruct((b, s, K), jnp.int32),
    )(cxT, cyT, czT, xs[:, None, :], ys[:, None, :], zs[:, None, :])


def _mlp_body(S, K, centered, g_ref, wt_ref, bias_ref,
              cx_ref, cy_ref, cz_ref,
              pmax_ref, pmin_ref, sh_ref, sh2_ref):
    c = g_ref.shape[2]
    co = wt_ref.shape[1]
    g3 = g_ref[0].reshape(S, K, c)
    if centered:
        ctr = jnp.concatenate(
            [cx_ref[0], cy_ref[0], cz_ref[0],
             jnp.zeros((S, c - 3), jnp.float32)], axis=1)
        g3 = g3 - ctr[:, None, :]
    gb = g3.astype(jnp.bfloat16).reshape(S * K, c)
    wb = wt_ref[...].astype(jnp.bfloat16)
    h = jnp.dot(gb, wb, preferred_element_type=jnp.float32)
    h3 = h.reshape(S, K, co) + bias_ref[...][:, None, :]
    pmax_ref[0] = jnp.max(h3, axis=1)
    pmin_ref[0] = jnp.min(h3, axis=1)
    sh = jnp.sum(jnp.sum(h3, axis=1), axis=0, keepdims=True)
    sq = h3 * h3
    sh2 = jnp.sum(jnp.sum(sq, axis=1), axis=0, keepdims=True)

    @pl.when(pl.program_id(0) == 0)
    def _():
        sh_ref[...] = jnp.zeros_like(sh_ref)
        sh2_ref[...] = jnp.zeros_like(sh2_ref)

    sh_ref[...] += sh
    sh2_ref[...] += sh2


def _mlp_pool(g, wt, bias, cxT, cyT, czT, S, K, centered):
    b = g.shape[0]
    c = g.shape[2]
    co = wt.shape[1]
    cspec = pl.BlockSpec((1, S, 1), lambda i: (i, 0, 0))
    acc = pl.BlockSpec((1, co), lambda i: (0, 0))
    return pl.pallas_call(
        functools.partial(_mlp_body, S, K, centered),
        grid=(b,),
        in_specs=[
            pl.BlockSpec((1, S * K, c), lambda i: (i, 0, 0)),
            pl.BlockSpec((c, co), lambda i: (0, 0)),
            pl.BlockSpec((1, co), lambda i: (0, 0)),
            cspec, cspec, cspec,
        ],
        out_specs=[
            pl.BlockSpec((1, S, co), lambda i: (i, 0, 0)),
            pl.BlockSpec((1, S, co), lambda i: (i, 0, 0)),
            acc, acc,
        ],
        out_shape=[
            jax.ShapeDtypeStruct((b, S, co), jnp.float32),
            jax.ShapeDtypeStruct((b, S, co), jnp.float32),
            jax.ShapeDtypeStruct((1, co), jnp.float32),
            jax.ShapeDtypeStruct((1, co), jnp.float32),
        ],
    )(g, wt, bias, cxT, cyT, czT)


def _bn_body(cnt, ct, pmax_ref, pmin_ref, sh_ref, sh2_ref, gam_ref,
             bet_ref, cx_ref, cy_ref, cz_ref, tbl_ref):
    s, co = pmax_ref.shape[1], pmax_ref.shape[2]
    mu = sh_ref[...] * (1.0 / cnt)
    var = jnp.maximum(sh2_ref[...] * (1.0 / cnt) - mu * mu, 0.0)
    inv = 1.0 / jnp.sqrt(var + 1e-5)
    a = inv * gam_ref[...]
    cc = bet_ref[...] - mu * a
    sel = jnp.where(a >= 0.0, pmax_ref[0], pmin_ref[0])
    y = jnp.maximum(sel * a + cc, 0.0)
    tbl = jnp.concatenate(
        [cx_ref[0], cy_ref[0], cz_ref[0], y,
         jnp.zeros((s, ct - 3 - co), jnp.float32)], axis=1)
    tbl_ref[0] = tbl


def _bn_table(pmax, pmin, sh, sh2, gam, bet, cxT, cyT, czT, cnt, ct):
    b, s, co = pmax.shape
    cspec = pl.BlockSpec((1, s, 1), lambda i: (i, 0, 0))
    stat = pl.BlockSpec((1, co), lambda i: (0, 0))
    return pl.pallas_call(
        functools.partial(_bn_body, cnt, ct),
        grid=(b,),
        in_specs=[
            pl.BlockSpec((1, s, co), lambda i: (i, 0, 0)),
            pl.BlockSpec((1, s, co), lambda i: (i, 0, 0)),
            stat, stat, stat, stat,
            cspec, cspec, cspec,
        ],
        out_specs=pl.BlockSpec((1, s, ct), lambda i: (i, 0, 0)),
        out_shape=jax.ShapeDtypeStruct((b, s, ct), jnp.float32),
    )(pmax, pmin, sh, sh2, gam, bet, cxT, cyT, czT)


def _final_body(cnt, pmax_ref, pmin_ref, sh_ref, sh2_ref, gam_ref, bet_ref,
                wfct_ref, bfc_ref, out_ref):
    mu = sh_ref[...] * (1.0 / cnt)
    var = jnp.maximum(sh2_ref[...] * (1.0 / cnt) - mu * mu, 0.0)
    inv = 1.0 / jnp.sqrt(var + 1e-5)
    a = inv * gam_ref[...]
    cc = bet_ref[...] - mu * a
    sel = jnp.where(a >= 0.0, pmax_ref[:, 0, :], pmin_ref[:, 0, :])
    y = jnp.maximum(sel * a + cc, 0.0)
    out_ref[...] = (jnp.dot(y.astype(jnp.bfloat16),
                            wfct_ref[...].astype(jnp.bfloat16),
                            preferred_element_type=jnp.float32)
                    + bfc_ref[...])


def _final(pmax, pmin, sh, sh2, gam, bet, wfct, bfc):
    b = pmax.shape[0]
    cf = wfct.shape[1]
    return pl.pallas_call(
        functools.partial(_final_body, CNT3),
        out_shape=jax.ShapeDtypeStruct((b, cf), jnp.float32),
    )(pmax, pmin, sh, sh2, gam, bet, wfct, bfc)


def _gather_rows(tbl2d, idx, chunk):
    b, s, k = idx.shape
    m = b * s * k
    c = tbl2d.shape[1]
    info = plsc.get_sparse_core_info()
    nw = info.num_cores * info.num_subcores
    per_w = m // nw
    nch = per_w // chunk
    assert per_w * nw == m and nch * chunk == per_w and nch % 2 == 0
    mesh = plsc.VectorSubcoreMesh(core_axis_name="c", subcore_axis_name="s")

    @functools.partial(
        pl.kernel, mesh=mesh,
        out_type=jax.ShapeDtypeStruct((m, c), jnp.float32),
        compiler_params=pltpu.CompilerParams(use_tc_tiling_on_sc=False),
        scratch_types=[
            pltpu.VMEM((per_w,), jnp.int32),
            pltpu.VMEM((chunk, c), jnp.float32),
            pltpu.VMEM((chunk, c), jnp.float32),
            pltpu.SemaphoreType.DMA,
            pltpu.SemaphoreType.DMA,
        ],
    )
    def kern(tbl_hbm, idx_hbm, out_hbm, idx_v, rows0, rows1, sem0, sem1):
        wid = lax.axis_index("s") * info.num_cores + lax.axis_index("c")
        base = wid * per_w
        pltpu.sync_copy(idx_hbm.at[pl.ds(base, per_w)], idx_v)
        pltpu.async_copy(tbl_hbm.at[idx_v.at[pl.ds(0, chunk)]], rows0,
                         sem0)

        def body(j2, carry):
            j = j2 * 2
            pltpu.async_copy(
                tbl_hbm.at[idx_v.at[pl.ds((j + 1) * chunk, chunk)]],
                rows1, sem1)
            pltpu.make_async_copy(
                tbl_hbm.at[idx_v.at[pl.ds(0, chunk)]], rows0, sem0).wait()
            pltpu.sync_copy(rows0, out_hbm.at[pl.ds(base + j * chunk,
                                                    chunk)])

            @pl.when(j + 2 < nch)
            def _():
                pltpu.async_copy(
                    tbl_hbm.at[idx_v.at[pl.ds((j + 2) * chunk, chunk)]],
                    rows0, sem0)

            pltpu.make_async_copy(
                tbl_hbm.at[idx_v.at[pl.ds(0, chunk)]], rows1, sem1).wait()
            pltpu.sync_copy(rows1, out_hbm.at[pl.ds(base + (j + 1) * chunk,
                                                    chunk)])
            return carry

        lax.fori_loop(0, nch // 2, body, 0)

    out = kern(tbl2d, idx.reshape(m))
    return out.reshape(b, s * k, c)


def kernel(xyz, W1, b1, g1, be1, W2, b2, g2, be2, W3, b3, g3, be3, Wfc, bfc):
    f32 = jnp.float32
    xs = xyz[:, :, 0]
    ys = xyz[:, :, 1]
    zs = xyz[:, :, 2]

    c1x, c1y, c1z = _fps(xs, ys, zs, S1)
    c1xT, c1yT, c1zT = c1x[:, :, None], c1y[:, :, None], c1z[:, :, None]
    idx1 = _ball_query(c1xT, c1yT, c1zT, xs, ys, zs, K1, R1)

    tbl0 = jnp.zeros((B, N1, C1T), f32).at[:, :, 0].set(xs)
    tbl0 = tbl0.at[:, :, 1].set(ys).at[:, :, 2].set(zs)
    g1rows = _gather_rows(tbl0.reshape(B * N1, C1T), idx1, chunk=1024)

    wt1 = jnp.zeros((C1T, 128), f32).at[0:3, :].set(W1.T)
    p1max, p1min, sh1, sh1sq = _mlp_pool(
        g1rows, wt1, b1[None, :], c1xT, c1yT, c1zT, S1, K1, True)
    tbl1 = _bn_table(p1max, p1min, sh1, sh1sq, g1[None, :], be1[None, :],
                     c1xT, c1yT, c1zT, CNT1, C2T)

    c2x, c2y, c2z = _fps(c1x, c1y, c1z, S2)
    c2xT, c2yT, c2zT = c2x[:, :, None], c2y[:, :, None], c2z[:, :, None]
    idx2 = _ball_query(c2xT, c2yT, c2zT, c1x, c1y, c1z, K2, R2)
    g2rows = _gather_rows(tbl1.reshape(B * S1, C2T), idx2, chunk=128)

    wt2 = jnp.zeros((C2T, 256), f32).at[0:131, :].set(W2.T)
    p2max, p2min, sh2, sh2sq = _mlp_pool(
        g2rows, wt2, b2[None, :], c2xT, c2yT, c2zT, S2, K2, True)
    tbl2 = _bn_table(p2max, p2min, sh2, sh2sq, g2[None, :], be2[None, :],
                     c2xT, c2yT, c2zT, CNT2, C3T)

    wt3 = jnp.zeros((C3T, 512), f32).at[0:259, :].set(W3.T)
    zcol = jnp.zeros((B, 1, 1), f32)
    p3max, p3min, sh3, sh3sq = _mlp_pool(
        tbl2, wt3, b3[None, :], zcol, zcol, zcol, 1, S2, False)
    out = _final(p3max, p3min, sh3, sh3sq, g3[None, :], be3[None, :],
                 Wfc.T, bfc[None, :])
    return out


# speedup vs baseline: 19.7188x; 1.4710x over previous
import functools

import jax
import jax.numpy as jnp
from jax import lax
from jax.experimental import pallas as pl
from jax.experimental.pallas import tpu as pltpu
from jax.experimental.pallas import tpu_sc as plsc

B = 16
N1, S1, K1, R1 = 4096, 512, 32, 0.2
S2, K2, R2 = 128, 64, 0.4
C1T = 16
C2T = 144
C3T = 272
CNT1 = B * S1 * K1
CNT2 = B * S2 * K2
CNT3 = B * S2


def _fps_body(npoint, x_ref, y_ref, z_ref, cx_ref, cy_ref, cz_ref):
    b, n = x_ref.shape
    xs = x_ref[...]
    ys = y_ref[...]
    zs = z_ref[...]
    iota_n = jax.lax.broadcasted_iota(jnp.int32, (b, n), 1)
    iota_s = jax.lax.broadcasted_iota(jnp.int32, (b, npoint), 1)

    def body(i, carry):
        dist, far, cxh, cyh, czh = carry
        mask = iota_n == far
        cx = jnp.sum(jnp.where(mask, xs, 0.0), axis=1, keepdims=True)
        cy = jnp.sum(jnp.where(mask, ys, 0.0), axis=1, keepdims=True)
        cz = jnp.sum(jnp.where(mask, zs, 0.0), axis=1, keepdims=True)
        hit = iota_s == i
        cxh = jnp.where(hit, cx, cxh)
        cyh = jnp.where(hit, cy, cyh)
        czh = jnp.where(hit, cz, czh)
        dx = xs - cx
        dy = ys - cy
        dz = zs - cz
        d = (dx * dx + dy * dy) + dz * dz
        dist = jnp.minimum(dist, d)
        m = jnp.max(dist, axis=1, keepdims=True)
        far = jnp.min(jnp.where(dist == m, iota_n, n), axis=1, keepdims=True)
        return dist, far, cxh, cyh, czh

    init = (
        jnp.full((b, n), 1e10, jnp.float32),
        jnp.zeros((b, 1), jnp.int32),
        jnp.zeros((b, npoint), jnp.float32),
        jnp.zeros((b, npoint), jnp.float32),
        jnp.zeros((b, npoint), jnp.float32),
    )
    _, _, cxh, cyh, czh = jax.lax.fori_loop(0, npoint, body, init)
    cx_ref[...] = cxh
    cy_ref[...] = cyh
    cz_ref[...] = czh


def _fps(xs, ys, zs, npoint):
    b, n = xs.shape
    out = jax.ShapeDtypeStruct((b, npoint), jnp.float32)
    return pl.pallas_call(
        functools.partial(_fps_body, npoint),
        out_shape=[out, out, out],
    )(xs, ys, zs)


def _bq_body(K, r2, npts, cx_ref, cy_ref, cz_ref, x_ref, y_ref, z_ref,
             out_ref):
    s = cx_ref.shape[1]
    n = x_ref.shape[2]
    cxs = cx_ref[0]
    cys = cy_ref[0]
    czs = cz_ref[0]
    px = x_ref[0]
    py = y_ref[0]
    pz = z_ref[0]
    bf = jnp.bfloat16
    bcx = cxs.astype(bf).astype(jnp.float32)
    bcy = cys.astype(bf).astype(jnp.float32)
    bcz = czs.astype(bf).astype(jnp.float32)
    bpx = px.astype(bf).astype(jnp.float32)
    bpy = py.astype(bf).astype(jnp.float32)
    bpz = pz.astype(bf).astype(jnp.float32)
    cross = (bcx * bpx + bcz * bpz) + bcy * bpy
    s2 = (cxs * cxs + czs * czs) + cys * cys
    p2 = (px * px + pz * pz) + py * py
    d = (-2.0 * cross + s2) + p2
    iota = jax.lax.broadcasted_iota(jnp.int32, (s, n), 1)
    cand = jnp.where(jnp.logical_not(d > r2), iota, n)
    iota_k = jax.lax.broadcasted_iota(jnp.int32, (s, K), 1)
    first = jnp.min(cand, axis=1, keepdims=True)
    out0 = jnp.broadcast_to(first, (s, K))

    def cond(state):
        k, prev, _ = state
        return jnp.logical_and(k < K, jnp.min(prev) < n)

    def body(state):
        k, prev, out = state
        val = jnp.where(cand > prev, cand, n)
        cur = jnp.min(val, axis=1, keepdims=True)
        colv = jnp.where(cur == n, first, cur)
        out = jnp.where(iota_k == k, colv, out)
        return k + 1, cur, out

    _, _, out = jax.lax.while_loop(cond, body, (jnp.int32(1), first, out0))
    out = jnp.minimum(out, n - 1)
    out_ref[0] = out + pl.program_id(0) * npts


def _ball_query(cxT, cyT, czT, xs, ys, zs, K, radius):
    s = cxT.shape[1]
    b, n = xs.shape
    cspec = pl.BlockSpec((1, s, 1), lambda i: (i, 0, 0))
    pspec = pl.BlockSpec((1, 1, n), lambda i: (i, 0, 0))
    return pl.pallas_call(
        functools.partial(_bq_body, K, radius * radius, n),
        grid=(b,),
        in_specs=[cspec, cspec, cspec, pspec, pspec, pspec],
        out_specs=pl.BlockSpec((1, s, K), lambda i: (i, 0, 0)),
        out_shape=jax.ShapeDtypeStruct((b, s, K), jnp.int32),
    )(cxT, cyT, czT, xs[:, None, :], ys[:, None, :], zs[:, None, :])


def _mlp_body(S, K, centered, g_ref, wt_ref, bias_ref,
              cx_ref, cy_ref, cz_ref,
              pmax_ref, pmin_ref, sh_ref, sh2_ref):
    c = g_ref.shape[2]
    co = wt_ref.shape[1]
    g3 = g_ref[0].reshape(S, K, c)
    if centered:
        ctr = jnp.concatenate(
            [cx_ref[0], cy_ref[0], cz_ref[0],
             jnp.zeros((S, c - 3), jnp.float32)], axis=1)
        g3 = g3 - ctr[:, None, :]
    gb = g3.astype(jnp.bfloat16).reshape(S * K, c)
    wb = wt_ref[...].astype(jnp.bfloat16)
    h = jnp.dot(gb, wb, preferred_element_type=jnp.float32)
    h3 = h.reshape(S, K, co) + bias_ref[...][:, None, :]
    pmax_ref[0] = jnp.max(h3, axis=1)
    pmin_ref[0] = jnp.min(h3, axis=1)
    sh = jnp.sum(jnp.sum(h3, axis=1), axis=0, keepdims=True)
    sq = h3 * h3
    sh2 = jnp.sum(jnp.sum(sq, axis=1), axis=0, keepdims=True)

    @pl.when(pl.program_id(0) == 0)
    def _():
        sh_ref[...] = jnp.zeros_like(sh_ref)
        sh2_ref[...] = jnp.zeros_like(sh2_ref)

    sh_ref[...] += sh
    sh2_ref[...] += sh2


def _mlp_pool(g, wt, bias, cxT, cyT, czT, S, K, centered):
    b = g.shape[0]
    c = g.shape[2]
    co = wt.shape[1]
    cspec = pl.BlockSpec((1, S, 1), lambda i: (i, 0, 0))
    acc = pl.BlockSpec((1, co), lambda i: (0, 0))
    return pl.pallas_call(
        functools.partial(_mlp_body, S, K, centered),
        grid=(b,),
        in_specs=[
            pl.BlockSpec((1, S * K, c), lambda i: (i, 0, 0)),
            pl.BlockSpec((c, co), lambda i: (0, 0)),
            pl.BlockSpec((1, co), lambda i: (0, 0)),
            cspec, cspec, cspec,
        ],
        out_specs=[
            pl.BlockSpec((1, S, co), lambda i: (i, 0, 0)),
            pl.BlockSpec((1, S, co), lambda i: (i, 0, 0)),
            acc, acc,
        ],
        out_shape=[
            jax.ShapeDtypeStruct((b, S, co), jnp.float32),
            jax.ShapeDtypeStruct((b, S, co), jnp.float32),
            jax.ShapeDtypeStruct((1, co), jnp.float32),
            jax.ShapeDtypeStruct((1, co), jnp.float32),
        ],
    )(g, wt, bias, cxT, cyT, czT)


def _bn_body(cnt, ct, pmax_ref, pmin_ref, sh_ref, sh2_ref, gam_ref,
             bet_ref, cx_ref, cy_ref, cz_ref, tbl_ref):
    s, co = pmax_ref.shape[1], pmax_ref.shape[2]
    mu = sh_ref[...] * (1.0 / cnt)
    var = jnp.maximum(sh2_ref[...] * (1.0 / cnt) - mu * mu, 0.0)
    inv = 1.0 / jnp.sqrt(var + 1e-5)
    a = inv * gam_ref[...]
    cc = bet_ref[...] - mu * a
    sel = jnp.where(a >= 0.0, pmax_ref[0], pmin_ref[0])
    y = jnp.maximum(sel * a + cc, 0.0)
    tbl = jnp.concatenate(
        [cx_ref[0], cy_ref[0], cz_ref[0], y,
         jnp.zeros((s, ct - 3 - co), jnp.float32)], axis=1)
    tbl_ref[0] = tbl


def _bn_table(pmax, pmin, sh, sh2, gam, bet, cxT, cyT, czT, cnt, ct):
    b, s, co = pmax.shape
    cspec = pl.BlockSpec((1, s, 1), lambda i: (i, 0, 0))
    stat = pl.BlockSpec((1, co), lambda i: (0, 0))
    return pl.pallas_call(
        functools.partial(_bn_body, cnt, ct),
        grid=(b,),
        in_specs=[
            pl.BlockSpec((1, s, co), lambda i: (i, 0, 0)),
            pl.BlockSpec((1, s, co), lambda i: (i, 0, 0)),
            stat, stat, stat, stat,
            cspec, cspec, cspec,
        ],
        out_specs=pl.BlockSpec((1, s, ct), lambda i: (i, 0, 0)),
        out_shape=jax.ShapeDtypeStruct((b, s, ct), jnp.float32),
    )(pmax, pmin, sh, sh2, gam, bet, cxT, cyT, czT)


def _final_body(cnt, pmax_ref, pmin_ref, sh_ref, sh2_ref, gam_ref, bet_ref,
                wfct_ref, bfc_ref, out_ref):
    mu = sh_ref[...] * (1.0 / cnt)
    var = jnp.maximum(sh2_ref[...] * (1.0 / cnt) - mu * mu, 0.0)
    inv = 1.0 / jnp.sqrt(var + 1e-5)
    a = inv * gam_ref[...]
    cc = bet_ref[...] - mu * a
    sel = jnp.where(a >= 0.0, pmax_ref[:, 0, :], pmin_ref[:, 0, :])
    y = jnp.maximum(sel * a + cc, 0.0)
    out_ref[...] = (jnp.dot(y.astype(jnp.bfloat16),
                            wfct_ref[...].astype(jnp.bfloat16),
                            preferred_element_type=jnp.float32)
                    + bfc_ref[...])


def _final(pmax, pmin, sh, sh2, gam, bet, wfct, bfc):
    b = pmax.shape[0]
    cf = wfct.shape[1]
    return pl.pallas_call(
        functools.partial(_final_body, CNT3),
        out_shape=jax.ShapeDtypeStruct((b, cf), jnp.float32),
    )(pmax, pmin, sh, sh2, gam, bet, wfct, bfc)


def _gather_rows(tbl2d, idx, chunk):
    b, s, k = idx.shape
    m = b * s * k
    c = tbl2d.shape[1]
    info = plsc.get_sparse_core_info()
    nw = info.num_cores * info.num_subcores
    per_w = m // nw
    nch = per_w // chunk
    assert per_w * nw == m and nch * chunk == per_w and nch % 2 == 0
    mesh = plsc.VectorSubcoreMesh(core_axis_name="c", subcore_axis_name="s")

    @functools.partial(
        pl.kernel, mesh=mesh,
        out_type=jax.ShapeDtypeStruct((m, c), jnp.float32),
        compiler_params=pltpu.CompilerParams(use_tc_tiling_on_sc=False),
        scratch_types=[
            pltpu.VMEM((per_w,), jnp.int32),
            pltpu.VMEM((chunk, c), jnp.float32),
            pltpu.VMEM((chunk, c), jnp.float32),
            pltpu.SemaphoreType.DMA,
            pltpu.SemaphoreType.DMA,
        ],
    )
    def kern(tbl_hbm, idx_hbm, out_hbm, idx_v, rows0, rows1, sem0, sem1):
        wid = lax.axis_index("s") * info.num_cores + lax.axis_index("c")
        base = wid * per_w
        pltpu.sync_copy(idx_hbm.at[pl.ds(base, per_w)], idx_v)
        pltpu.async_copy(tbl_hbm.at[idx_v.at[pl.ds(0, chunk)]], rows0,
                         sem0)

        def body(j2, carry):
            j = j2 * 2
            pltpu.async_copy(
                tbl_hbm.at[idx_v.at[pl.ds((j + 1) * chunk, chunk)]],
                rows1, sem1)
            pltpu.make_async_copy(
                tbl_hbm.at[idx_v.at[pl.ds(0, chunk)]], rows0, sem0).wait()
            pltpu.sync_copy(rows0, out_hbm.at[pl.ds(base + j * chunk,
                                                    chunk)])

            @pl.when(j + 2 < nch)
            def _():
                pltpu.async_copy(
                    tbl_hbm.at[idx_v.at[pl.ds((j + 2) * chunk, chunk)]],
                    rows0, sem0)

            pltpu.make_async_copy(
                tbl_hbm.at[idx_v.at[pl.ds(0, chunk)]], rows1, sem1).wait()
            pltpu.sync_copy(rows1, out_hbm.at[pl.ds(base + (j + 1) * chunk,
                                                    chunk)])
            return carry

        lax.fori_loop(0, nch // 2, body, 0)

    out = kern(tbl2d, idx.reshape(m))
    return out.reshape(b, s * k, c)


def kernel(xyz, W1, b1, g1, be1, W2, b2, g2, be2, W3, b3, g3, be3, Wfc, bfc):
    f32 = jnp.float32
    xs = xyz[:, :, 0]
    ys = xyz[:, :, 1]
    zs = xyz[:, :, 2]

    c1x, c1y, c1z = _fps(xs, ys, zs, S1)
    c1xT, c1yT, c1zT = c1x[:, :, None], c1y[:, :, None], c1z[:, :, None]
    idx1 = _ball_query(c1xT, c1yT, c1zT, xs, ys, zs, K1, R1)

    tbl0 = jnp.zeros((B, N1, C1T), f32).at[:, :, 0].set(xs)
    tbl0 = tbl0.at[:, :, 1].set(ys).at[:, :, 2].set(zs)
    g1rows = _gather_rows(tbl0.reshape(B * N1, C1T), idx1, chunk=1024)

    wt1 = jnp.zeros((C1T, 128), f32).at[0:3, :].set(W1.T)
    p1max, p1min, sh1, sh1sq = _mlp_pool(
        g1rows, wt1, b1[None, :], c1xT, c1yT, c1zT, S1, K1, True)
    tbl1 = _bn_table(p1max, p1min, sh1, sh1sq, g1[None, :], be1[None, :],
                     c1xT, c1yT, c1zT, CNT1, C2T)

    c2x, c2y, c2z = _fps(c1x, c1y, c1z, S2)
    c2xT, c2yT, c2zT = c2x[:, :, None], c2y[:, :, None], c2z[:, :, None]
    idx2 = _ball_query(c2xT, c2yT, c2zT, c1x, c1y, c1z, K2, R2)
    g2rows = _gather_rows(tbl1.reshape(B * S1, C2T), idx2, chunk=128)

    wt2 = jnp.zeros((C2T, 256), f32).at[0:131, :].set(W2.T)
    p2max, p2min, sh2, sh2sq = _mlp_pool(
        g2rows, wt2, b2[None, :], c2xT, c2yT, c2zT, S2, K2, True)
    tbl2 = _bn_table(p2max, p2min, sh2, sh2sq, g2[None, :], be2[None, :],
                     c2xT, c2yT, c2zT, CNT2, C3T)

    wt3 = jnp.zeros((C3T, 512), f32).at[0:259, :].set(W3.T)
    zcol = jnp.zeros((B, 1, 1), f32)
    p3max, p3min, sh3, sh3sq = _mlp_pool(
        tbl2, wt3, b3[None, :], zcol, zcol, zcol, 1, S2, False)
    out = _final(p3max, p3min, sh3, sh3sq, g3[None, :], be3[None, :],
                 Wfc.T, bfc[None, :])
    return out
